# baseline (speedup 1.0000x reference)
import functools

import jax
import jax.numpy as jnp
from jax.experimental import pallas as pl
from jax.experimental.pallas import tpu as pltpu

CTX = 96
CAND_BLK = 1024
GROUP = 128
Q_BLK = 64

_f1 = pl.BlockSpec(None, lambda i: (0,))
_f2 = pl.BlockSpec(None, lambda i: (0, 0))


def _norm_scale(v):
    t = v + 1e-5
    return jax.lax.rsqrt(t) * t


def _xblock_kernel(xn, W_lin, b_lin, W0a, b0a, W0b, b0b, x_out):
    x = jnp.dot(xn[...], W_lin[...], preferred_element_type=jnp.float32) + b_lin[...]
    h = jnp.maximum(jnp.dot(x, W0a[...], preferred_element_type=jnp.float32) + b0a[...], 0.0)
    h = jnp.dot(h, W0b[...], preferred_element_type=jnp.float32) + b0b[...]
    x_out[...] = x + h


def _encode_x(xn, W_lin, b_lin, W0a, b0a, W0b, b0b):
    n, d_in = xn.shape
    d_main = W_lin.shape[1]
    blk = CAND_BLK if n % CAND_BLK == 0 else n
    return pl.pallas_call(
        _xblock_kernel,
        grid=(n // blk,),
        in_specs=[pl.BlockSpec((blk, d_in), lambda i: (i, 0)),
                  _f2, _f1, _f2, _f1, _f2, _f1],
        out_specs=pl.BlockSpec((blk, d_main), lambda i: (i, 0)),
        out_shape=jax.ShapeDtypeStruct((n, d_main), jnp.float32),
    )(xn, W_lin, b_lin, W0a, b0a, W0b, b0b)


def _qk_kernel(xq, mq, vq, g_mix, be_mix, W_k, b_k, kq_out, knorm_out):
    xs = (xq[...] - mq[...]) / _norm_scale(vq[...]) * g_mix[...] + be_mix[...]
    k = jnp.dot(xs, W_k[...], preferred_element_type=jnp.float32) + b_k[...]
    kq_out[...] = k
    knorm_out[...] = jnp.sum(k * k, axis=-1, keepdims=True)


def _query_k(xq, mq, vq, g_mix, be_mix, W_k, b_k):
    B, d_main = xq.shape
    return pl.pallas_call(
        _qk_kernel,
        out_shape=(jax.ShapeDtypeStruct((B, d_main), jnp.float32),
                   jax.ShapeDtypeStruct((B, 1), jnp.float32)),
    )(xq, mq, vq, g_mix, be_mix, W_k, b_k)


def _ck_kernel(n_valid, xc, mc, vc, kq, knorm, g_mix, be_mix, W_k, b_k,
               ck_out, s_out, m_out):
    i = pl.program_id(0)
    xs = (xc[...] - mc[...]) / _norm_scale(vc[...]) * g_mix[...] + be_mix[...]
    ck = jnp.dot(xs, W_k[...], preferred_element_type=jnp.float32) + b_k[...]
    ck_out[...] = ck
    cknorm = jnp.sum(ck * ck, axis=-1)[None, :]
    dot = jnp.dot(kq[...], ck.T, preferred_element_type=jnp.float32)
    d2 = (knorm[...] - 2.0 * dot) + cknorm
    s = -d2
    col = i * CAND_BLK + jax.lax.broadcasted_iota(jnp.int32, (1, CAND_BLK), 1)
    s = jnp.where(col < n_valid, s, -jnp.inf)
    s_out[...] = s
    B = s.shape[0]
    m_out[...] = jnp.max(s.reshape(B, CAND_BLK // GROUP, GROUP),
                         axis=-1)[None, ...]


def _cand_scores(n_valid, xc, mc, vc, kq, knorm, g_mix, be_mix, W_k, b_k):
    npad, d_main = xc.shape
    B = kq.shape[0]
    nblk = npad // CAND_BLK
    return pl.pallas_call(
        functools.partial(_ck_kernel, n_valid),
        grid=(nblk,),
        in_specs=[
            pl.BlockSpec((CAND_BLK, d_main), lambda i: (i, 0)),
            pl.BlockSpec((CAND_BLK, 1), lambda i: (i, 0)),
            pl.BlockSpec((CAND_BLK, 1), lambda i: (i, 0)),
            _f2, _f2, _f1, _f1, _f2, _f1,
        ],
        out_specs=(
            pl.BlockSpec((CAND_BLK, d_main), lambda i: (i, 0)),
            pl.BlockSpec((B, CAND_BLK), lambda i: (0, i)),
            pl.BlockSpec((1, B, CAND_BLK // GROUP), lambda i: (i, 0, 0)),
        ),
        out_shape=(
            jax.ShapeDtypeStruct((npad, d_main), jnp.float32),
            jax.ShapeDtypeStruct((B, npad), jnp.float32),
            jax.ShapeDtypeStruct((nblk, B, CAND_BLK // GROUP), jnp.float32),
        ),
    )(xc, mc, vc, kq, knorm, g_mix, be_mix, W_k, b_k)


def _ln_tail(x, g, b):
    rn = 1.0 / x.shape[-1]
    m = jnp.sum(x, axis=-1, keepdims=True) * rn
    d = x - m
    v = jnp.sum(d * d, axis=-1, keepdims=True) * rn
    return d / _norm_scale(v) * g + b


def _tail_kernel(kq, xq, svals, ctx_k, cy, W_lab, b_lab, W_ta, b_ta, W_tb,
                 g1, be1, W1a, b1a, W1b, b1b, gh, beh, W_head, b_head,
                 y_out, p_out):
    k = kq[...]
    x = xq[...]
    s = svals[...]
    ckv = ctx_k[...]
    Q, D = k.shape

    smax = jnp.max(s, axis=-1, keepdims=True)
    e = jnp.exp(s - smax)
    probs = e / jnp.sum(e, axis=-1, keepdims=True)
    p_out[...] = probs

    diff = (k[:, None, :] - ckv).reshape(Q * CTX, D)
    t1 = jnp.maximum(
        jnp.dot(diff, W_ta[...], preferred_element_type=jnp.float32)
        + b_ta[...], 0.0)
    tv = jnp.dot(t1, W_tb[...], preferred_element_type=jnp.float32)
    yemb = (cy[...][:, :, None] * W_lab[...][0][None, None, :]
            + b_lab[...][None, None, :])
    values = yemb + tv.reshape(Q, CTX, D)
    context_x = jnp.sum(probs[:, :, None] * values, axis=1)
    x = x + context_x

    h = _ln_tail(x, g1[...], be1[...])
    h = jnp.maximum(
        jnp.dot(h, W1a[...], preferred_element_type=jnp.float32) + b1a[...],
        0.0)
    h = jnp.dot(h, W1b[...], preferred_element_type=jnp.float32) + b1b[...]
    x = x + h
    xh = jnp.maximum(_ln_tail(x, gh[...], beh[...]), 0.0)
    y_out[...] = (jnp.dot(xh, W_head[...], preferred_element_type=jnp.float32)
                  + b_head[...])


def _tail(kq, xq, svals, ctx_k, cy, W_lab, b_lab, W_ta, b_ta, W_tb, g1, be1,
          W1a, b1a, W1b, b1b, gh, beh, W_head, b_head):
    B, d_main = kq.shape
    qb = min(Q_BLK, B)
    return pl.pallas_call(
        _tail_kernel,
        grid=(B // qb,),
        in_specs=[
            pl.BlockSpec((qb, d_main), lambda i: (i, 0)),
            pl.BlockSpec((qb, d_main), lambda i: (i, 0)),
            pl.BlockSpec((qb, CTX), lambda i: (i, 0)),
            pl.BlockSpec((qb, CTX, d_main), lambda i: (i, 0, 0)),
            pl.BlockSpec((qb, CTX), lambda i: (i, 0)),
            _f2, _f1, _f2, _f1, _f2, _f1, _f1, _f2, _f1, _f2, _f1, _f1,
            _f1, _f2, _f1,
        ],
        out_specs=(
            pl.BlockSpec((qb, 1), lambda i: (i, 0)),
            pl.BlockSpec((qb, CTX), lambda i: (i, 0)),
        ),
        out_shape=(
            jax.ShapeDtypeStruct((B, 1), jnp.float32),
            jax.ShapeDtypeStruct((B, CTX), jnp.float32),
        ),
    )(kq, xq, svals, ctx_k, cy, W_lab, b_lab, W_ta, b_ta, W_tb, g1, be1,
      W1a, b1a, W1b, b1b, gh, beh, W_head, b_head)


def kernel(x_num, candidate_x_num, candidate_y, candidate_idx, context_size,
           W_lin, b_lin, W0a, b0a, W0b, b0b, g_mix, be_mix, W_k, b_k,
           W_lab, b_lab, W_ta, b_ta, W_tb, g1, be1, W1a, b1a, W1b, b1b,
           gh, beh, W_head, b_head):
    B = x_num.shape[0]
    N = candidate_x_num.shape[0]
    npad = ((N + CAND_BLK - 1) // CAND_BLK) * CAND_BLK

    xq = _encode_x(x_num, W_lin, b_lin, W0a, b0a, W0b, b0b)
    xc = _encode_x(jnp.pad(candidate_x_num, ((0, npad - N), (0, 0))),
                   W_lin, b_lin, W0a, b0a, W0b, b0b)

    def _x_clone(xn):
        x = xn @ W_lin + b_lin
        h = jax.nn.relu(x @ W0a + b0a) @ W0b + b0b
        return x + h
    xq_c = _x_clone(x_num)
    xc_c = _x_clone(candidate_x_num)
    mq = jnp.mean(xq_c, axis=-1, keepdims=True)
    vq = jnp.var(xq_c, axis=-1, keepdims=True)
    mc_v = jnp.mean(xc_c, axis=-1, keepdims=True)
    vc_v = jnp.var(xc_c, axis=-1, keepdims=True)
    mc = jnp.pad(mc_v, ((0, npad - N), (0, 0)))
    vc = jnp.pad(vc_v, ((0, npad - N), (0, 0)))

    kq, knorm = _query_k(xq, mq, vq, g_mix, be_mix, W_k, b_k)
    ck, S, M = _cand_scores(N, xc, mc, vc, kq, knorm, g_mix, be_mix, W_k, b_k)
    del M

    svals, context_idx = jax.lax.top_k(S, CTX)
    ctx_k = ck[context_idx]
    cy = candidate_y[context_idx]
    absolute_context_idx = candidate_idx[context_idx]

    y_pred, probs = _tail(kq, xq, svals, ctx_k, cy, W_lab, b_lab, W_ta, b_ta,
                          W_tb, g1, be1, W1a, b1a, W1b, b1b, gh, beh,
                          W_head, b_head)


    context_misses = (jnp.zeros((B,), dtype=jnp.float32)
                      + jnp.asarray(context_size - CTX, dtype=jnp.float32))
    return (y_pred, absolute_context_idx, probs, context_misses)


# speedup vs baseline: 3.0266x; 3.0266x over previous
import functools

import jax
import jax.numpy as jnp
from jax.experimental import pallas as pl
from jax.experimental.pallas import tpu as pltpu
from jax.experimental.pallas import tpu_sc as plsc

CTX = 96
CAND_BLK = 1024
GROUP = 128
Q_BLK = 64

_f1 = pl.BlockSpec(None, lambda i: (0,))
_f2 = pl.BlockSpec(None, lambda i: (0, 0))


def _norm_scale(v):
    t = v + 1e-5
    return jax.lax.rsqrt(t) * t


def _xblock_kernel(xn, W_lin, b_lin, W0a, b0a, W0b, b0b, x_out):
    x = jnp.dot(xn[...], W_lin[...], preferred_element_type=jnp.float32) + b_lin[...]
    h = jnp.maximum(jnp.dot(x, W0a[...], preferred_element_type=jnp.float32) + b0a[...], 0.0)
    h = jnp.dot(h, W0b[...], preferred_element_type=jnp.float32) + b0b[...]
    x_out[...] = x + h


def _encode_x(xn, W_lin, b_lin, W0a, b0a, W0b, b0b):
    n, d_in = xn.shape
    d_main = W_lin.shape[1]
    blk = CAND_BLK if n % CAND_BLK == 0 else n
    return pl.pallas_call(
        _xblock_kernel,
        grid=(n // blk,),
        in_specs=[pl.BlockSpec((blk, d_in), lambda i: (i, 0)),
                  _f2, _f1, _f2, _f1, _f2, _f1],
        out_specs=pl.BlockSpec((blk, d_main), lambda i: (i, 0)),
        out_shape=jax.ShapeDtypeStruct((n, d_main), jnp.float32),
    )(xn, W_lin, b_lin, W0a, b0a, W0b, b0b)


def _qk_kernel(xq, mq, vq, g_mix, be_mix, W_k, b_k, kq_out, knorm_out):
    xs = (xq[...] - mq[...]) / _norm_scale(vq[...]) * g_mix[...] + be_mix[...]
    k = jnp.dot(xs, W_k[...], preferred_element_type=jnp.float32) + b_k[...]
    kq_out[...] = k
    knorm_out[...] = jnp.sum(k * k, axis=-1, keepdims=True)


def _query_k(xq, mq, vq, g_mix, be_mix, W_k, b_k):
    B, d_main = xq.shape
    return pl.pallas_call(
        _qk_kernel,
        out_shape=(jax.ShapeDtypeStruct((B, d_main), jnp.float32),
                   jax.ShapeDtypeStruct((B, 1), jnp.float32)),
    )(xq, mq, vq, g_mix, be_mix, W_k, b_k)


def _ck_kernel(n_valid, xc, mc, vc, kq, knorm, g_mix, be_mix, W_k, b_k,
               ck_out, s_out, m_out):
    i = pl.program_id(0)
    xs = (xc[...] - mc[...]) / _norm_scale(vc[...]) * g_mix[...] + be_mix[...]
    ck = jnp.dot(xs, W_k[...], preferred_element_type=jnp.float32) + b_k[...]
    ck_out[...] = ck
    cknorm = jnp.sum(ck * ck, axis=-1)[None, :]
    dot = jnp.dot(kq[...], ck.T, preferred_element_type=jnp.float32)
    d2 = (knorm[...] - 2.0 * dot) + cknorm
    s = -d2
    col = i * CAND_BLK + jax.lax.broadcasted_iota(jnp.int32, (1, CAND_BLK), 1)
    s = jnp.where(col < n_valid, s, -jnp.inf)
    s_out[...] = s
    B = s.shape[0]
    m_out[...] = jnp.max(s.reshape(B, CAND_BLK // GROUP, GROUP),
                         axis=-1)[None, ...]


def _cand_scores(n_valid, xc, mc, vc, kq, knorm, g_mix, be_mix, W_k, b_k):
    npad, d_main = xc.shape
    B = kq.shape[0]
    nblk = npad // CAND_BLK
    return pl.pallas_call(
        functools.partial(_ck_kernel, n_valid),
        grid=(nblk,),
        in_specs=[
            pl.BlockSpec((CAND_BLK, d_main), lambda i: (i, 0)),
            pl.BlockSpec((CAND_BLK, 1), lambda i: (i, 0)),
            pl.BlockSpec((CAND_BLK, 1), lambda i: (i, 0)),
            _f2, _f2, _f1, _f1, _f2, _f1,
        ],
        out_specs=(
            pl.BlockSpec((CAND_BLK, d_main), lambda i: (i, 0)),
            pl.BlockSpec((B, CAND_BLK), lambda i: (0, i)),
            pl.BlockSpec((1, B, CAND_BLK // GROUP), lambda i: (i, 0, 0)),
        ),
        out_shape=(
            jax.ShapeDtypeStruct((npad, d_main), jnp.float32),
            jax.ShapeDtypeStruct((B, npad), jnp.float32),
            jax.ShapeDtypeStruct((nblk, B, CAND_BLK // GROUP), jnp.float32),
        ),
    )(xc, mc, vc, kq, knorm, g_mix, be_mix, W_k, b_k)


def _ln_tail(x, g, b):
    rn = 1.0 / x.shape[-1]
    m = jnp.sum(x, axis=-1, keepdims=True) * rn
    d = x - m
    v = jnp.sum(d * d, axis=-1, keepdims=True) * rn
    return d / _norm_scale(v) * g + b


def _tail_kernel(kq, xq, svals, ctx_k, cy, W_lab, b_lab, W_ta, b_ta, W_tb,
                 g1, be1, W1a, b1a, W1b, b1b, gh, beh, W_head, b_head,
                 y_out, p_out):
    k = kq[...]
    x = xq[...]
    s = svals[...]
    ckv = ctx_k[...]
    Q, D = k.shape

    smax = jnp.max(s, axis=-1, keepdims=True)
    e = jnp.exp(s - smax)
    probs = e / jnp.sum(e, axis=-1, keepdims=True)
    p_out[...] = probs

    diff = (k[:, None, :] - ckv).reshape(Q * CTX, D)
    t1 = jnp.maximum(
        jnp.dot(diff, W_ta[...], preferred_element_type=jnp.float32)
        + b_ta[...], 0.0)
    tv = jnp.dot(t1, W_tb[...], preferred_element_type=jnp.float32)
    yemb = (cy[...][:, :, None] * W_lab[...][0][None, None, :]
            + b_lab[...][None, None, :])
    values = yemb + tv.reshape(Q, CTX, D)
    context_x = jnp.sum(probs[:, :, None] * values, axis=1)
    x = x + context_x

    h = _ln_tail(x, g1[...], be1[...])
    h = jnp.maximum(
        jnp.dot(h, W1a[...], preferred_element_type=jnp.float32) + b1a[...],
        0.0)
    h = jnp.dot(h, W1b[...], preferred_element_type=jnp.float32) + b1b[...]
    x = x + h
    xh = jnp.maximum(_ln_tail(x, gh[...], beh[...]), 0.0)
    y_out[...] = (jnp.dot(xh, W_head[...], preferred_element_type=jnp.float32)
                  + b_head[...])


def _tail(kq, xq, svals, ctx_k, cy, W_lab, b_lab, W_ta, b_ta, W_tb, g1, be1,
          W1a, b1a, W1b, b1b, gh, beh, W_head, b_head):
    B, d_main = kq.shape
    qb = min(Q_BLK, B)
    return pl.pallas_call(
        _tail_kernel,
        grid=(B // qb,),
        in_specs=[
            pl.BlockSpec((qb, d_main), lambda i: (i, 0)),
            pl.BlockSpec((qb, d_main), lambda i: (i, 0)),
            pl.BlockSpec((qb, CTX), lambda i: (i, 0)),
            pl.BlockSpec((qb, CTX, d_main), lambda i: (i, 0, 0)),
            pl.BlockSpec((qb, CTX), lambda i: (i, 0)),
            _f2, _f1, _f2, _f1, _f2, _f1, _f1, _f2, _f1, _f2, _f1, _f1,
            _f1, _f2, _f1,
        ],
        out_specs=(
            pl.BlockSpec((qb, 1), lambda i: (i, 0)),
            pl.BlockSpec((qb, CTX), lambda i: (i, 0)),
        ),
        out_shape=(
            jax.ShapeDtypeStruct((B, 1), jnp.float32),
            jax.ShapeDtypeStruct((B, CTX), jnp.float32),
        ),
    )(kq, xq, svals, ctx_k, cy, W_lab, b_lab, W_ta, b_ta, W_tb, g1, be1,
      W1a, b1a, W1b, b1b, gh, beh, W_head, b_head)


CAP = 768
SVW = 896
NGRP = 128
GQ = 32

NEG_INF = float("-inf")


def _dgather(x, idx):
    return jax.lax.gather(
        x, idx[:, None],
        jax.lax.GatherDimensionNumbers(offset_dims=(),
                                       collapsed_slice_dims=(0,),
                                       start_index_map=(0,)),
        (1,), mode=jax.lax.GatherScatterMode.PROMISE_IN_BOUNDS)


def _prefix16(x, lanes):
    for d in (1, 2, 4, 8):
        sh = _dgather(x, jnp.maximum(lanes - d, 0))
        x = x + jnp.where(lanes >= d, sh, 0)
    return x


def _sc_collect(S4, agid, tau_b):
    B = tau_b.shape[0]
    mesh = plsc.VectorSubcoreMesh(core_axis_name="c", subcore_axis_name="s")

    @functools.partial(
        pl.kernel,
        mesh=mesh,
        out_type=(
            jax.ShapeDtypeStruct((B, SVW), jnp.float32),
            jax.ShapeDtypeStruct((B, SVW), jnp.int32),
        ),
        scratch_types=[
            pltpu.VMEM((NGRP,), jnp.int32),
            pltpu.VMEM((NGRP, 128), jnp.float32),
            pltpu.VMEM((SVW,), jnp.float32),
            pltpu.VMEM((SVW,), jnp.int32),
            pltpu.VMEM((16,), jnp.float32),
            pltpu.SemaphoreType.DMA,
        ],
    )
    def k(S4_h, agid_h, tau_h, outv_h, outi_h,
          agid_v, grp_v, vals_v, idx_v, tau_v, sem):
        wid = jax.lax.axis_index("s") * 2 + jax.lax.axis_index("c")
        row0 = wid * GQ

        def row_body(i):
            row = row0 + i
            pltpu.sync_copy(agid_h.at[row], agid_v)
            pltpu.sync_copy(tau_h.at[row], tau_v)
            pltpu.async_copy(S4_h.at[agid_v], grp_v, sem).wait()
            neg = jnp.full((16,), NEG_INF, dtype=jnp.float32)
            zero = jnp.zeros((16,), dtype=jnp.int32)
            for t in range(SVW // 16):
                vals_v[pl.ds(t * 16, 16)] = neg
                idx_v[pl.ds(t * 16, 16)] = zero
            tau_r = tau_v[...]
            lanes = jax.lax.iota(jnp.int32, 16)
            lane15 = jnp.full((16,), 15, jnp.int32)
            cap_v = jnp.full((16,), CAP, jnp.int32)

            @plsc.parallel_loop(0, NGRP, carry=jnp.zeros((16,), jnp.int32))
            def grp_body(g, off_v):
                v = grp_v[g, pl.ds(0, 16)]
                mask = v >= tau_r
                cum = _prefix16(mask.astype(jnp.int32), lanes)
                vals_v[pl.ds(0, 16)] = cum.astype(jnp.float32)
                return off_v + cum
            pltpu.sync_copy(vals_v, outv_h.at[row])
            pltpu.sync_copy(idx_v, outi_h.at[row])

        plsc.parallel_loop(0, GQ)(row_body)

    return k(S4, agid, tau_b)


def _order_kernel(sv, si, v_out, i_out):
    v = sv[...]
    ix = si[...]
    Q = v.shape[0]
    rank = jnp.zeros((Q, SVW), jnp.int32)
    for c in range(SVW // 128):
        vc = v[:, c * 128:(c + 1) * 128]
        ic = ix[:, c * 128:(c + 1) * 128]
        gt = vc[:, :, None] > v[:, None, :]
        tie = jnp.logical_and(vc[:, :, None] == v[:, None, :],
                              ic[:, :, None] < ix[:, None, :])
        rank = rank + jnp.sum(jnp.logical_or(gt, tie).astype(jnp.int32),
                              axis=1)
    r96 = jax.lax.broadcasted_iota(jnp.int32, (1, 1, CTX), 2)
    oh = rank[:, :, None] == r96
    v_out[...] = jnp.sum(jnp.where(oh, v[:, :, None], 0.0), axis=1)
    i_out[...] = jnp.sum(jnp.where(oh, ix[:, :, None], 0), axis=1)


def _order(sv, si):
    B = sv.shape[0]
    qb = 8
    return pl.pallas_call(
        _order_kernel,
        grid=(B // qb,),
        in_specs=[pl.BlockSpec((qb, SVW), lambda i: (i, 0)),
                  pl.BlockSpec((qb, SVW), lambda i: (i, 0))],
        out_specs=(pl.BlockSpec((qb, CTX), lambda i: (i, 0)),
                   pl.BlockSpec((qb, CTX), lambda i: (i, 0))),
        out_shape=(jax.ShapeDtypeStruct((B, CTX), jnp.float32),
                   jax.ShapeDtypeStruct((B, CTX), jnp.int32)),
    )(sv, si)


def kernel(x_num, candidate_x_num, candidate_y, candidate_idx, context_size,
           W_lin, b_lin, W0a, b0a, W0b, b0b, g_mix, be_mix, W_k, b_k,
           W_lab, b_lab, W_ta, b_ta, W_tb, g1, be1, W1a, b1a, W1b, b1b,
           gh, beh, W_head, b_head):
    B = x_num.shape[0]
    N = candidate_x_num.shape[0]
    npad = ((N + CAND_BLK - 1) // CAND_BLK) * CAND_BLK

    xq = _encode_x(x_num, W_lin, b_lin, W0a, b0a, W0b, b0b)
    xc = _encode_x(jnp.pad(candidate_x_num, ((0, npad - N), (0, 0))),
                   W_lin, b_lin, W0a, b0a, W0b, b0b)

    def _x_clone(xn):
        x = xn @ W_lin + b_lin
        h = jax.nn.relu(x @ W0a + b0a) @ W0b + b0b
        return x + h
    xq_c = _x_clone(x_num)
    xc_c = _x_clone(candidate_x_num)
    mq = jnp.mean(xq_c, axis=-1, keepdims=True)
    vq = jnp.var(xq_c, axis=-1, keepdims=True)
    mc_v = jnp.mean(xc_c, axis=-1, keepdims=True)
    vc_v = jnp.var(xc_c, axis=-1, keepdims=True)
    mc = jnp.pad(mc_v, ((0, npad - N), (0, 0)))
    vc = jnp.pad(vc_v, ((0, npad - N), (0, 0)))

    kq, knorm = _query_k(xq, mq, vq, g_mix, be_mix, W_k, b_k)
    ck, S, M = _cand_scores(N, xc, mc, vc, kq, knorm, g_mix, be_mix, W_k, b_k)

    M2 = M.transpose(1, 0, 2).reshape(B, npad // GROUP)
    gvals, gidx = jax.lax.top_k(M2, NGRP)
    S3 = S.reshape(B, npad // GROUP, GROUP)
    Sg = jnp.take_along_axis(S3, gidx[:, :, None], axis=1).reshape(
        B, NGRP * GROUP)
    svals, pos = jax.lax.top_k(Sg, CTX)
    context_idx = (jnp.take_along_axis(gidx, pos >> 7, axis=1) * GROUP
                   + (pos & (GROUP - 1)))
    ctx_k = ck[context_idx]
    cy = candidate_y[context_idx]
    absolute_context_idx = candidate_idx[context_idx]

    y_pred, probs = _tail(kq, xq, svals, ctx_k, cy, W_lab, b_lab, W_ta, b_ta,
                          W_tb, g1, be1, W1a, b1a, W1b, b1b, gh, beh,
                          W_head, b_head)


    context_misses = (jnp.zeros((B,), dtype=jnp.float32)
                      + jnp.asarray(context_size - CTX, dtype=jnp.float32))
    return (y_pred, absolute_context_idx, probs, context_misses)


# speedup vs baseline: 9.8433x; 3.2523x over previous
import functools

import jax
import jax.numpy as jnp
from jax.experimental import pallas as pl
from jax.experimental.pallas import tpu as pltpu
from jax.experimental.pallas import tpu_sc as plsc

CTX = 96
CAND_BLK = 1024
GROUP = 128
Q_BLK = 64

_f1 = pl.BlockSpec(None, lambda i: (0,))
_f2 = pl.BlockSpec(None, lambda i: (0, 0))


def _norm_scale(v):
    t = v + 1e-5
    return jax.lax.rsqrt(t) * t


def _xblock_kernel(xn, W_lin, b_lin, W0a, b0a, W0b, b0b, x_out):
    x = jnp.dot(xn[...], W_lin[...], preferred_element_type=jnp.float32) + b_lin[...]
    h = jnp.maximum(jnp.dot(x, W0a[...], preferred_element_type=jnp.float32) + b0a[...], 0.0)
    h = jnp.dot(h, W0b[...], preferred_element_type=jnp.float32) + b0b[...]
    x_out[...] = x + h


def _encode_x(xn, W_lin, b_lin, W0a, b0a, W0b, b0b):
    n, d_in = xn.shape
    d_main = W_lin.shape[1]
    blk = CAND_BLK if n % CAND_BLK == 0 else n
    return pl.pallas_call(
        _xblock_kernel,
        grid=(n // blk,),
        in_specs=[pl.BlockSpec((blk, d_in), lambda i: (i, 0)),
                  _f2, _f1, _f2, _f1, _f2, _f1],
        out_specs=pl.BlockSpec((blk, d_main), lambda i: (i, 0)),
        out_shape=jax.ShapeDtypeStruct((n, d_main), jnp.float32),
    )(xn, W_lin, b_lin, W0a, b0a, W0b, b0b)


def _qk_kernel(xq, mq, vq, g_mix, be_mix, W_k, b_k, kq_out, knorm_out):
    xs = (xq[...] - mq[...]) / _norm_scale(vq[...]) * g_mix[...] + be_mix[...]
    k = jnp.dot(xs, W_k[...], preferred_element_type=jnp.float32) + b_k[...]
    kq_out[...] = k
    knorm_out[...] = jnp.sum(k * k, axis=-1, keepdims=True)


def _query_k(xq, mq, vq, g_mix, be_mix, W_k, b_k):
    B, d_main = xq.shape
    return pl.pallas_call(
        _qk_kernel,
        out_shape=(jax.ShapeDtypeStruct((B, d_main), jnp.float32),
                   jax.ShapeDtypeStruct((B, 1), jnp.float32)),
    )(xq, mq, vq, g_mix, be_mix, W_k, b_k)


def _ck_kernel(n_valid, xc, mc, vc, kq, knorm, g_mix, be_mix, W_k, b_k,
               ck_out, s_out, m_out):
    i = pl.program_id(0)
    xs = (xc[...] - mc[...]) / _norm_scale(vc[...]) * g_mix[...] + be_mix[...]
    ck = jnp.dot(xs, W_k[...], preferred_element_type=jnp.float32) + b_k[...]
    ck_out[...] = ck
    cknorm = jnp.sum(ck * ck, axis=-1)[None, :]
    dot = jnp.dot(kq[...], ck.T, preferred_element_type=jnp.float32)
    d2 = (knorm[...] - 2.0 * dot) + cknorm
    s = -d2
    col = i * CAND_BLK + jax.lax.broadcasted_iota(jnp.int32, (1, CAND_BLK), 1)
    s = jnp.where(col < n_valid, s, -jnp.inf)
    s_out[...] = s
    B = s.shape[0]
    m_out[...] = jnp.max(s.reshape(B, CAND_BLK // GROUP, GROUP),
                         axis=-1)[None, ...]


def _cand_scores(n_valid, xc, mc, vc, kq, knorm, g_mix, be_mix, W_k, b_k):
    npad, d_main = xc.shape
    B = kq.shape[0]
    nblk = npad // CAND_BLK
    return pl.pallas_call(
        functools.partial(_ck_kernel, n_valid),
        grid=(nblk,),
        in_specs=[
            pl.BlockSpec((CAND_BLK, d_main), lambda i: (i, 0)),
            pl.BlockSpec((CAND_BLK, 1), lambda i: (i, 0)),
            pl.BlockSpec((CAND_BLK, 1), lambda i: (i, 0)),
            _f2, _f2, _f1, _f1, _f2, _f1,
        ],
        out_specs=(
            pl.BlockSpec((CAND_BLK, d_main), lambda i: (i, 0)),
            pl.BlockSpec((B, CAND_BLK), lambda i: (0, i)),
            pl.BlockSpec((1, B, CAND_BLK // GROUP), lambda i: (i, 0, 0)),
        ),
        out_shape=(
            jax.ShapeDtypeStruct((npad, d_main), jnp.float32),
            jax.ShapeDtypeStruct((B, npad), jnp.float32),
            jax.ShapeDtypeStruct((nblk, B, CAND_BLK // GROUP), jnp.float32),
        ),
    )(xc, mc, vc, kq, knorm, g_mix, be_mix, W_k, b_k)


def _ln_tail(x, g, b):
    rn = 1.0 / x.shape[-1]
    m = jnp.sum(x, axis=-1, keepdims=True) * rn
    d = x - m
    v = jnp.sum(d * d, axis=-1, keepdims=True) * rn
    return d / _norm_scale(v) * g + b


def _tail_kernel(kq, xq, svals, ctx_k, cy, W_lab, b_lab, W_ta, b_ta, W_tb,
                 g1, be1, W1a, b1a, W1b, b1b, gh, beh, W_head, b_head,
                 y_out, p_out):
    k = kq[...]
    x = xq[...]
    s = svals[...]
    ckv = ctx_k[...]
    Q, D = k.shape

    smax = jnp.max(s, axis=-1, keepdims=True)
    e = jnp.exp(s - smax)
    probs = e / jnp.sum(e, axis=-1, keepdims=True)
    p_out[...] = probs

    diff = (k[:, None, :] - ckv).reshape(Q * CTX, D)
    t1 = jnp.maximum(
        jnp.dot(diff, W_ta[...], preferred_element_type=jnp.float32)
        + b_ta[...], 0.0)
    tv = jnp.dot(t1, W_tb[...], preferred_element_type=jnp.float32)
    yemb = (cy[...][:, :, None] * W_lab[...][0][None, None, :]
            + b_lab[...][None, None, :])
    values = yemb + tv.reshape(Q, CTX, D)
    context_x = jnp.sum(probs[:, :, None] * values, axis=1)
    x = x + context_x

    h = _ln_tail(x, g1[...], be1[...])
    h = jnp.maximum(
        jnp.dot(h, W1a[...], preferred_element_type=jnp.float32) + b1a[...],
        0.0)
    h = jnp.dot(h, W1b[...], preferred_element_type=jnp.float32) + b1b[...]
    x = x + h
    xh = jnp.maximum(_ln_tail(x, gh[...], beh[...]), 0.0)
    y_out[...] = (jnp.dot(xh, W_head[...], preferred_element_type=jnp.float32)
                  + b_head[...])


def _tail(kq, xq, svals, ctx_k, cy, W_lab, b_lab, W_ta, b_ta, W_tb, g1, be1,
          W1a, b1a, W1b, b1b, gh, beh, W_head, b_head):
    B, d_main = kq.shape
    qb = min(Q_BLK, B)
    return pl.pallas_call(
        _tail_kernel,
        grid=(B // qb,),
        in_specs=[
            pl.BlockSpec((qb, d_main), lambda i: (i, 0)),
            pl.BlockSpec((qb, d_main), lambda i: (i, 0)),
            pl.BlockSpec((qb, CTX), lambda i: (i, 0)),
            pl.BlockSpec((qb, CTX, d_main), lambda i: (i, 0, 0)),
            pl.BlockSpec((qb, CTX), lambda i: (i, 0)),
            _f2, _f1, _f2, _f1, _f2, _f1, _f1, _f2, _f1, _f2, _f1, _f1,
            _f1, _f2, _f1,
        ],
        out_specs=(
            pl.BlockSpec((qb, 1), lambda i: (i, 0)),
            pl.BlockSpec((qb, CTX), lambda i: (i, 0)),
        ),
        out_shape=(
            jax.ShapeDtypeStruct((B, 1), jnp.float32),
            jax.ShapeDtypeStruct((B, CTX), jnp.float32),
        ),
    )(kq, xq, svals, ctx_k, cy, W_lab, b_lab, W_ta, b_ta, W_tb, g1, be1,
      W1a, b1a, W1b, b1b, gh, beh, W_head, b_head)


CAP = 768
SVW = 896
NGRP = 128
GQ = 32

NEG_INF = float("-inf")


def _dgather(x, idx):
    return jax.lax.gather(
        x, idx[:, None],
        jax.lax.GatherDimensionNumbers(offset_dims=(),
                                       collapsed_slice_dims=(0,),
                                       start_index_map=(0,)),
        (1,), mode=jax.lax.GatherScatterMode.PROMISE_IN_BOUNDS)


def _prefix16(x, lanes):
    for d in (1, 2, 4, 8):
        sh = _dgather(x, jnp.maximum(lanes - d, 0))
        x = x + jnp.where(lanes >= d, sh, 0)
    return x


def _sc_collect(S4, agid, tau_b):
    B = tau_b.shape[0]
    mesh = plsc.VectorSubcoreMesh(core_axis_name="c", subcore_axis_name="s")

    @functools.partial(
        pl.kernel,
        mesh=mesh,
        out_type=(
            jax.ShapeDtypeStruct((B, SVW), jnp.float32),
            jax.ShapeDtypeStruct((B, SVW), jnp.int32),
        ),
        scratch_types=[
            pltpu.VMEM((NGRP,), jnp.int32),
            pltpu.VMEM((NGRP, 128), jnp.float32),
            pltpu.VMEM((SVW,), jnp.float32),
            pltpu.VMEM((SVW,), jnp.int32),
            pltpu.VMEM((16,), jnp.float32),
            pltpu.SemaphoreType.DMA,
        ],
    )
    def k(S4_h, agid_h, tau_h, outv_h, outi_h,
          agid_v, grp_v, vals_v, idx_v, tau_v, sem):
        wid = jax.lax.axis_index("s") * 2 + jax.lax.axis_index("c")
        row0 = wid * GQ

        def row_body(i):
            row = row0 + i
            pltpu.sync_copy(agid_h.at[row], agid_v)
            pltpu.sync_copy(tau_h.at[row], tau_v)
            pltpu.async_copy(S4_h.at[agid_v], grp_v, sem).wait()
            neg = jnp.full((16,), NEG_INF, dtype=jnp.float32)
            zero = jnp.zeros((16,), dtype=jnp.int32)
            for t in range(SVW // 16):
                vals_v[pl.ds(t * 16, 16)] = neg
                idx_v[pl.ds(t * 16, 16)] = zero
            tau_r = tau_v[...]
            lanes = jax.lax.iota(jnp.int32, 16)
            lane15 = jnp.full((16,), 15, jnp.int32)
            cap_v = jnp.full((16,), CAP, jnp.int32)

            @plsc.parallel_loop(0, NGRP, carry=jnp.zeros((16,), jnp.int32))
            def grp_body(g, off_v):
                v = grp_v[g, pl.ds(0, 16)]
                mask = v >= tau_r
                cum = _prefix16(mask.astype(jnp.int32), lanes)
                vals_v[pl.ds(0, 16)] = cum.astype(jnp.float32)
                return off_v + cum
            pltpu.sync_copy(vals_v, outv_h.at[row])
            pltpu.sync_copy(idx_v, outi_h.at[row])

        plsc.parallel_loop(0, GQ)(row_body)

    return k(S4, agid, tau_b)


def _order_kernel(sv, si, v_out, i_out):
    v = sv[...]
    ix = si[...]
    Q = v.shape[0]
    rank = jnp.zeros((Q, SVW), jnp.int32)
    for c in range(SVW // 128):
        vc = v[:, c * 128:(c + 1) * 128]
        ic = ix[:, c * 128:(c + 1) * 128]
        gt = vc[:, :, None] > v[:, None, :]
        tie = jnp.logical_and(vc[:, :, None] == v[:, None, :],
                              ic[:, :, None] < ix[:, None, :])
        rank = rank + jnp.sum(jnp.logical_or(gt, tie).astype(jnp.int32),
                              axis=1)
    r96 = jax.lax.broadcasted_iota(jnp.int32, (1, 1, CTX), 2)
    oh = rank[:, :, None] == r96
    v_out[...] = jnp.sum(jnp.where(oh, v[:, :, None], 0.0), axis=1)
    i_out[...] = jnp.sum(jnp.where(oh, ix[:, :, None], 0), axis=1)


def _order(sv, si):
    B = sv.shape[0]
    qb = 8
    return pl.pallas_call(
        _order_kernel,
        grid=(B // qb,),
        in_specs=[pl.BlockSpec((qb, SVW), lambda i: (i, 0)),
                  pl.BlockSpec((qb, SVW), lambda i: (i, 0))],
        out_specs=(pl.BlockSpec((qb, CTX), lambda i: (i, 0)),
                   pl.BlockSpec((qb, CTX), lambda i: (i, 0))),
        out_shape=(jax.ShapeDtypeStruct((B, CTX), jnp.float32),
                   jax.ShapeDtypeStruct((B, CTX), jnp.int32)),
    )(sv, si)


def kernel(x_num, candidate_x_num, candidate_y, candidate_idx, context_size,
           W_lin, b_lin, W0a, b0a, W0b, b0b, g_mix, be_mix, W_k, b_k,
           W_lab, b_lab, W_ta, b_ta, W_tb, g1, be1, W1a, b1a, W1b, b1b,
           gh, beh, W_head, b_head):
    B = x_num.shape[0]
    N = candidate_x_num.shape[0]
    npad = ((N + CAND_BLK - 1) // CAND_BLK) * CAND_BLK

    xq = _encode_x(x_num, W_lin, b_lin, W0a, b0a, W0b, b0b)
    xc = _encode_x(jnp.pad(candidate_x_num, ((0, npad - N), (0, 0))),
                   W_lin, b_lin, W0a, b0a, W0b, b0b)

    def _x_clone(xn):
        x = xn @ W_lin + b_lin
        h = jax.nn.relu(x @ W0a + b0a) @ W0b + b0b
        return x + h
    xq_c = _x_clone(x_num)
    xc_c = _x_clone(candidate_x_num)
    mq = jnp.mean(xq_c, axis=-1, keepdims=True)
    vq = jnp.var(xq_c, axis=-1, keepdims=True)
    mc_v = jnp.mean(xc_c, axis=-1, keepdims=True)
    vc_v = jnp.var(xc_c, axis=-1, keepdims=True)
    mc = jnp.pad(mc_v, ((0, npad - N), (0, 0)))
    vc = jnp.pad(vc_v, ((0, npad - N), (0, 0)))

    kq, knorm = _query_k(xq, mq, vq, g_mix, be_mix, W_k, b_k)
    ck, S, M = _cand_scores(N, xc, mc, vc, kq, knorm, g_mix, be_mix, W_k, b_k)

    M2 = M.transpose(1, 0, 2).reshape(B, npad // GROUP)
    gvals, gidx = jax.lax.top_k(M2, NGRP)
    S3 = S.reshape(B, npad // GROUP, GROUP)
    Sg = jnp.take_along_axis(S3, gidx[:, :, None], axis=1)
    V = Sg.reshape(B, NGRP * 8, 16)
    M3 = jnp.max(V, axis=-1)
    _, gi2 = jax.lax.top_k(M3, NGRP)
    Sg2 = jnp.take_along_axis(V, gi2[:, :, None], axis=1).reshape(B, NGRP * 16)
    svals, p2 = jax.lax.top_k(Sg2, CTX)
    sub = jnp.take_along_axis(gi2, p2 >> 4, axis=1)
    pos = sub * 16 + (p2 & 15)
    context_idx = (jnp.take_along_axis(gidx, pos >> 7, axis=1) * GROUP
                   + (pos & (GROUP - 1)))
    ctx_k = ck[context_idx]
    cy = candidate_y[context_idx]
    absolute_context_idx = candidate_idx[context_idx]

    y_pred, probs = _tail(kq, xq, svals, ctx_k, cy, W_lab, b_lab, W_ta, b_ta,
                          W_tb, g1, be1, W1a, b1a, W1b, b1b, gh, beh,
                          W_head, b_head)


    context_misses = (jnp.zeros((B,), dtype=jnp.float32)
                      + jnp.asarray(context_size - CTX, dtype=jnp.float32))
    return (y_pred, absolute_context_idx, probs, context_misses)
